# Initial kernel scaffold; baseline (speedup 1.0000x reference)
import jax
import jax.numpy as jnp
from jax.experimental import pallas as pl


def kernel(A, x, movie_ids):
    raise NotImplementedError("write your pallas kernel here")


# speedup vs baseline: 3.4496x; 3.4496x over previous
import functools

import jax
import jax.numpy as jnp
from jax import lax
from jax.experimental import pallas as pl
from jax.experimental.pallas import tpu as pltpu
from jax.experimental.pallas import tpu_sc as plsc

VOCAB = 1000
EMB = 60
EMBP = 128
B = 16384
L = 20
TOTAL = B * L

_INFO = plsc.get_sparse_core_info()
NC = _INFO.num_cores
NS = _INFO.num_subcores
NW = NC * NS
PER_W = TOTAL // NW
SUB = 128
NSUB = 4
BLK = SUB * NSUB
IDX_ROWS = 8
SUPER = SUB * IDX_ROWS
NSUPER = PER_W // SUPER


def _matmul_body(a_ref, x_ref, o_ref):
    o_ref[...] = jnp.dot(a_ref[...], x_ref[...],
                         preferred_element_type=jnp.float32)


def _propagate(A, xp):
    return pl.pallas_call(
        _matmul_body,
        out_shape=jax.ShapeDtypeStruct((VOCAB, EMBP), jnp.float32),
    )(A, xp)


@functools.partial(
    pl.kernel,
    mesh=plsc.VectorSubcoreMesh(core_axis_name="c", subcore_axis_name="s"),
    out_type=jax.ShapeDtypeStruct((TOTAL, EMBP), jnp.float32),
    scratch_types=[
        pltpu.VMEM((IDX_ROWS, SUB), jnp.int32),
        pltpu.VMEM((BLK, EMBP), jnp.float32),
        pltpu.SemaphoreType.DMA,
    ],
)
def _gather(xg_hbm, idx_hbm, out_hbm, idx_v, rows_v, sem):
    wid = lax.axis_index("s") * NC + lax.axis_index("c")
    base_w = wid * PER_W

    def body(i, carry):
        sbase = pl.multiple_of(base_w + i * SUPER, SUPER)
        row0 = pl.multiple_of(sbase // SUB, IDX_ROWS)
        pltpu.sync_copy(idx_hbm.at[pl.ds(row0, IDX_ROWS)], idx_v)
        for half in range(SUPER // BLK):
            copies = []
            for j in range(NSUB):
                copies.append(pltpu.async_copy(
                    xg_hbm.at[idx_v.at[half * NSUB + j]],
                    rows_v.at[pl.ds(j * SUB, SUB)],
                    sem,
                ))
            for c in copies:
                c.wait()
            pltpu.sync_copy(
                rows_v, out_hbm.at[pl.ds(sbase + half * BLK, BLK)])
        return carry

    lax.fori_loop(0, NSUPER, body, 0)


def kernel(A, x, movie_ids):
    xp = jnp.pad(x, ((0, 0), (0, EMBP - EMB)))
    xg = _propagate(A, xp)
    idx = movie_ids.reshape(TOTAL // SUB, SUB).astype(jnp.int32)
    out = _gather(xg, idx)
    return out[:, :EMB].reshape(B, L, EMB)


# speedup vs baseline: 3.4548x; 1.0015x over previous
import functools

import jax
import jax.numpy as jnp
from jax import lax
from jax.experimental import pallas as pl
from jax.experimental.pallas import tpu as pltpu
from jax.experimental.pallas import tpu_sc as plsc

VOCAB = 1000
EMB = 60
EMBP = 128
B = 16384
L = 20
TOTAL = B * L

_INFO = plsc.get_sparse_core_info()
NC = _INFO.num_cores
NS = _INFO.num_subcores
NW = NC * NS
PER_W = TOTAL // NW
SUB = 128
NSUB = 2
BLK = SUB * NSUB
IDX_ROWS = 8
SUPER = SUB * IDX_ROWS
BPS = SUPER // BLK
NSUPER = PER_W // SUPER


def _matmul_body(a_ref, x_ref, o_ref):
    o_ref[...] = jnp.dot(a_ref[...], x_ref[...],
                         preferred_element_type=jnp.float32)


def _propagate(A, xp):
    return pl.pallas_call(
        _matmul_body,
        out_shape=jax.ShapeDtypeStruct((VOCAB, EMBP), jnp.float32),
    )(A, xp)


@functools.partial(
    pl.kernel,
    mesh=plsc.VectorSubcoreMesh(core_axis_name="c", subcore_axis_name="s"),
    out_type=jax.ShapeDtypeStruct((TOTAL, EMBP), jnp.float32),
    scratch_types=[
        pltpu.VMEM((2, IDX_ROWS, SUB), jnp.int32),
        pltpu.VMEM((2, BLK, EMBP), jnp.float32),
        [pltpu.SemaphoreType.DMA] * 2,
        [pltpu.SemaphoreType.DMA] * 2,
    ],
)
def _gather(xg_hbm, idx_hbm, out_hbm, idx_v, rows_v, sem_g, sem_w):
    wid = lax.axis_index("s") * NC + lax.axis_index("c")
    base_w = wid * PER_W
    row0_w = base_w // SUB

    def fire_gathers(r, slot, part):
        for j in range(NSUB):
            pltpu.async_copy(
                xg_hbm.at[idx_v.at[slot, part * NSUB + j]],
                rows_v.at[r, pl.ds(j * SUB, SUB)],
                sem_g[r],
            )

    def drain_gathers(r):
        for j in range(NSUB):
            pltpu.make_async_copy(
                xg_hbm.at[idx_v.at[0, 0]],
                rows_v.at[r, pl.ds(j * SUB, SUB)],
                sem_g[r],
            ).wait()

    def fire_write(r, s, k):
        pltpu.async_copy(
            rows_v.at[r],
            out_hbm.at[pl.ds(base_w + s * SUPER + k * BLK, BLK)],
            sem_w[r],
        )

    def drain_write(r):
        pltpu.make_async_copy(
            rows_v.at[r],
            out_hbm.at[pl.ds(base_w, BLK)],
            sem_w[r],
        ).wait()

    def fetch_idx(s, slot):
        pltpu.sync_copy(
            idx_hbm.at[pl.ds(pl.multiple_of(row0_w + s * IDX_ROWS, IDX_ROWS),
                             IDX_ROWS)],
            idx_v.at[slot],
        )

    fetch_idx(0, 0)
    fire_gathers(0, 0, 0)

    def body(s, carry):
        q = lax.rem(s, 2)
        qn = lax.rem(s + 1, 2)

        @pl.when(s < NSUPER - 1)
        def _():
            fetch_idx(s + 1, qn)

        for k in range(BPS):
            r = k % 2
            drain_gathers(r)
            if k == 0:
                @pl.when(s > 0)
                def _():
                    drain_write(1)
            else:
                drain_write(1 - r)
            if k < BPS - 1:
                fire_gathers(1 - r, q, k + 1)
            else:
                @pl.when(s < NSUPER - 1)
                def _():
                    fire_gathers(1 - r, qn, 0)
            fire_write(r, s, k)
        return carry

    lax.fori_loop(0, NSUPER, body, 0)
    drain_write(1)


def kernel(A, x, movie_ids):
    xp = jnp.pad(x, ((0, 0), (0, EMBP - EMB)))
    xg = _propagate(A, xp)
    idx = movie_ids.reshape(TOTAL // SUB, SUB).astype(jnp.int32)
    out = _gather(xg, idx)
    return out[:, :EMB].reshape(B, L, EMB)
